# baseline (speedup 1.0000x reference)
import functools

import jax
import jax.numpy as jnp
from jax import lax
from jax.experimental import pallas as pl
from jax.experimental.pallas import tpu as pltpu
from jax.experimental.pallas import tpu_sc as plsc

EMB = 16
CAT_FEATURES = 9
PP_FEATURES = 2
N_FEAT = CAT_FEATURES + PP_FEATURES
MAIN_DIM = CAT_FEATURES * EMB
PP_DIM = PP_FEATURES * EMB
GATHER_CHUNK = 128


@functools.partial(jax.jit, static_argnames=("batch", "emb",))
def _sc_gather(table, idx_fm, batch, emb):
    info = plsc.get_sparse_core_info()
    nw = info.num_cores * info.num_subcores
    b_per_w = batch // nw
    n_chunks = b_per_w // GATHER_CHUNK
    mesh = plsc.VectorSubcoreMesh(core_axis_name="c", subcore_axis_name="s")

    @functools.partial(
        pl.kernel,
        mesh=mesh,
        compiler_params=pltpu.CompilerParams(use_tc_tiling_on_sc=False),
        out_type=jax.ShapeDtypeStruct((batch, N_FEAT * emb), jnp.float32),
        scratch_types=[
            pltpu.VMEM((N_FEAT, b_per_w), jnp.int32),
            pltpu.VMEM((N_FEAT, b_per_w, emb), jnp.float32),
            pltpu.SemaphoreType.DMA,
            pltpu.SemaphoreType.DMA,
        ],
    )
    def gather_kernel(table_hbm, idx_hbm, out_hbm, idx_v, rows_v, gsem, osem):
        wid = lax.axis_index("s") * info.num_cores + lax.axis_index("c")
        base = wid * b_per_w

        for k in range(N_FEAT):
            pltpu.make_async_copy(
                idx_hbm.at[pl.ds(k * batch + base, b_per_w)],
                idx_v.at[k], osem,
            ).start()
        for k in range(N_FEAT):
            pltpu.make_async_copy(
                idx_hbm.at[pl.ds(k * batch + base, b_per_w)],
                idx_v.at[k], osem,
            ).wait()

        def start(j, carry):
            k = j // n_chunks
            off = (j % n_chunks) * GATHER_CHUNK
            pltpu.make_async_copy(
                table_hbm.at[idx_v.at[k].at[pl.ds(off, GATHER_CHUNK)]],
                rows_v.at[k].at[pl.ds(off, GATHER_CHUNK), :],
                gsem,
            ).start()
            return carry

        lax.fori_loop(0, N_FEAT * n_chunks, start, 0)

        def fin(j, carry):
            k = j // n_chunks
            off = (j % n_chunks) * GATHER_CHUNK
            pltpu.make_async_copy(
                table_hbm.at[idx_v.at[k].at[pl.ds(off, GATHER_CHUNK)]],
                rows_v.at[k].at[pl.ds(off, GATHER_CHUNK), :],
                gsem,
            ).wait()
            return carry

        lax.fori_loop(0, N_FEAT * n_chunks, fin, 0)

        for k in range(N_FEAT):
            pltpu.make_async_copy(
                rows_v.at[k],
                out_hbm.at[pl.ds(base, b_per_w), pl.ds(k * emb, emb)],
                osem,
            ).start()
        for k in range(N_FEAT):
            pltpu.make_async_copy(
                rows_v.at[k],
                out_hbm.at[pl.ds(base, b_per_w), pl.ds(k * emb, emb)],
                osem,
            ).wait()

    return gather_kernel(table, idx_fm)


def _dense_kernel(g_ref, w1_ref, b1_ref, w2_ref, b2_ref, wd_ref, bd_ref,
                  gw1a_ref, gw1b_ref, gb1_ref, gw2_ref, gb2_ref, c_ref,
                  out_ref):
    g = g_ref[...]
    pp = g[:, MAIN_DIM:]
    main = g[:, :MAIN_DIM]
    h = jnp.maximum(
        jnp.dot(pp, w1_ref[...], preferred_element_type=jnp.float32)
        + b1_ref[...], 0.0)
    e = 2.0 * jax.nn.sigmoid(
        jnp.dot(h, w2_ref[...], preferred_element_type=jnp.float32)
        + b2_ref[...])
    conc = e * main
    dense = jnp.dot(conc, wd_ref[...],
                    preferred_element_type=jnp.float32) + bd_ref[...]
    gh = jnp.maximum(
        jnp.dot(conc, gw1a_ref[...], preferred_element_type=jnp.float32)
        + jnp.dot(pp, gw1b_ref[...], preferred_element_type=jnp.float32)
        + gb1_ref[...], 0.0)
    gg = 2.0 * jax.nn.sigmoid(
        jnp.dot(gh, gw2_ref[...], preferred_element_type=jnp.float32)
        + gb2_ref[...])
    out_ref[...] = jax.nn.sigmoid(c_ref[...] * dense * gg)


def _dense_stage(gathered, w1, b1, w2, b2, wd, bd, gw1a, gw1b, gb1, gw2, gb2,
                 cvec):
    b = gathered.shape[0]
    blk = 2048
    grid = (b // blk,)

    def full(shape):
        return pl.BlockSpec(shape, lambda *_: (0,) * len(shape))

    return pl.pallas_call(
        _dense_kernel,
        grid=grid,
        in_specs=[
            pl.BlockSpec((blk, N_FEAT * EMB), lambda i: (i, 0)),
            full(w1.shape), full(b1.shape), full(w2.shape), full(b2.shape),
            full(wd.shape), full(bd.shape), full(gw1a.shape), full(gw1b.shape),
            full(gb1.shape), full(gw2.shape), full(gb2.shape), full(cvec.shape),
        ],
        out_specs=pl.BlockSpec((blk, 3), lambda i: (i, 0)),
        out_shape=jax.ShapeDtypeStruct((b, 3), jnp.float32),
    )(gathered, w1, b1, w2, b2, wd, bd, gw1a, gw1b, gb1, gw2, gb2, cvec)


def kernel(sdk_type, remote_host, device_type, dtu, click_goods_num,
           buy_click_num, goods_show_num, goods_click_num, brand_name,
           ppnet_cate1, ppnet_cate2, epnet_cate1, epnet_cate2,
           emb_table, gate_params, mlp_params):
    del epnet_cate1, epnet_cate2
    idx_fm = jnp.concatenate(
        [a.reshape(-1) for a in
         (sdk_type, remote_host, device_type, dtu, click_goods_num,
          buy_click_num, goods_show_num, goods_click_num, brand_name,
          ppnet_cate1, ppnet_cate2)])
    b = sdk_type.shape[0]

    gathered = _sc_gather(emb_table, idx_fm, batch=b, emb=EMB)

    w1 = jnp.concatenate([gp[0] for gp in gate_params], axis=1)
    b1 = jnp.concatenate([gp[1] for gp in gate_params]).reshape(1, -1)
    w2 = jax.scipy.linalg.block_diag(*[gp[2] for gp in gate_params])
    b2 = jnp.concatenate([gp[3] for gp in gate_params]).reshape(1, -1)

    wd = jnp.concatenate([mp['dense'][2][0] for mp in mlp_params], axis=1)
    bd = jnp.concatenate([mp['dense'][2][1] for mp in mlp_params]).reshape(1, -1)
    gw1 = jnp.concatenate([mp['gates'][2][0] for mp in mlp_params], axis=1)
    gw1a, gw1b = gw1[:MAIN_DIM], gw1[MAIN_DIM:]
    gb1 = jnp.concatenate([mp['gates'][2][1] for mp in mlp_params]).reshape(1, -1)
    gw2 = jax.scipy.linalg.block_diag(*[mp['gates'][2][2] for mp in mlp_params])
    gb2 = jnp.concatenate([mp['gates'][2][3] for mp in mlp_params]).reshape(1, -1)
    cvec = jnp.stack([mp['C'][2] for mp in mlp_params]).reshape(1, -1)

    out = _dense_stage(gathered, w1, b1, w2, b2, wd, bd, gw1a, gw1b, gb1,
                       gw2, gb2, cvec)
    return out.reshape(b, 3, 1)


# speedup vs baseline: 1.0009x; 1.0009x over previous
import functools

import jax
import jax.numpy as jnp
from jax import lax
from jax.experimental import pallas as pl
from jax.experimental.pallas import tpu as pltpu
from jax.experimental.pallas import tpu_sc as plsc

EMB = 16
CAT_FEATURES = 9
PP_FEATURES = 2
N_FEAT = CAT_FEATURES + PP_FEATURES
MAIN_DIM = CAT_FEATURES * EMB
PP_DIM = PP_FEATURES * EMB
GATHER_CHUNK = 128


@functools.partial(jax.jit, static_argnames=("batch", "emb",))
def _sc_gather(table, idx_t, batch, emb):
    info = plsc.get_sparse_core_info()
    nw = info.num_cores * info.num_subcores
    b_per_w = batch // nw
    n_chunks = b_per_w // GATHER_CHUNK
    mesh = plsc.VectorSubcoreMesh(core_axis_name="c", subcore_axis_name="s")

    @functools.partial(
        pl.kernel,
        mesh=mesh,
        compiler_params=pltpu.CompilerParams(use_tc_tiling_on_sc=False),
        out_type=jax.ShapeDtypeStruct((batch, N_FEAT * emb), jnp.float32),
        scratch_types=[
            pltpu.VMEM((N_FEAT, b_per_w), jnp.int32),
            pltpu.VMEM((N_FEAT, b_per_w, emb), jnp.float32),
            pltpu.SemaphoreType.DMA,
            pltpu.SemaphoreType.DMA,
        ],
    )
    def gather_kernel(table_hbm, idx_hbm, out_hbm, idx_v, rows_v,
                      gsem, osem):
        wid = lax.axis_index("s") * info.num_cores + lax.axis_index("c")
        base = wid * b_per_w
        pltpu.sync_copy(idx_hbm.at[:, pl.ds(base, b_per_w)], idx_v)

        def start(j, carry):
            k = j // n_chunks
            off = (j % n_chunks) * GATHER_CHUNK
            pltpu.make_async_copy(
                table_hbm.at[idx_v.at[k].at[pl.ds(off, GATHER_CHUNK)]],
                rows_v.at[k].at[pl.ds(off, GATHER_CHUNK), :],
                gsem,
            ).start()
            return carry

        lax.fori_loop(0, N_FEAT * n_chunks, start, 0)

        def fin(j, carry):
            k = j // n_chunks
            off = (j % n_chunks) * GATHER_CHUNK
            pltpu.make_async_copy(
                table_hbm.at[idx_v.at[k].at[pl.ds(off, GATHER_CHUNK)]],
                rows_v.at[k].at[pl.ds(off, GATHER_CHUNK), :],
                gsem,
            ).wait()
            return carry

        lax.fori_loop(0, N_FEAT * n_chunks, fin, 0)

        for k in range(N_FEAT):
            pltpu.make_async_copy(
                rows_v.at[k],
                out_hbm.at[pl.ds(base, b_per_w), pl.ds(k * emb, emb)],
                osem,
            ).start()
        for k in range(N_FEAT):
            pltpu.make_async_copy(
                rows_v.at[k],
                out_hbm.at[pl.ds(base, b_per_w), pl.ds(k * emb, emb)],
                osem,
            ).wait()

    return gather_kernel(table, idx_t)


def _dense_kernel(g_ref, w1_ref, b1_ref, w2_ref, b2_ref, wd_ref, bd_ref,
                  gw1a_ref, gw1b_ref, gb1_ref, gw2_ref, gb2_ref, c_ref,
                  out_ref):
    g = g_ref[...]
    pp = g[:, MAIN_DIM:]
    main = g[:, :MAIN_DIM]
    h = jnp.maximum(
        jnp.dot(pp, w1_ref[...], preferred_element_type=jnp.float32)
        + b1_ref[...], 0.0)
    e = 2.0 * jax.nn.sigmoid(
        jnp.dot(h, w2_ref[...], preferred_element_type=jnp.float32)
        + b2_ref[...])
    conc = e * main
    dense = jnp.dot(conc, wd_ref[...],
                    preferred_element_type=jnp.float32) + bd_ref[...]
    gh = jnp.maximum(
        jnp.dot(conc, gw1a_ref[...], preferred_element_type=jnp.float32)
        + jnp.dot(pp, gw1b_ref[...], preferred_element_type=jnp.float32)
        + gb1_ref[...], 0.0)
    gg = 2.0 * jax.nn.sigmoid(
        jnp.dot(gh, gw2_ref[...], preferred_element_type=jnp.float32)
        + gb2_ref[...])
    out_ref[...] = jax.nn.sigmoid(c_ref[...] * dense * gg)


def _dense_stage(gathered, w1, b1, w2, b2, wd, bd, gw1a, gw1b, gb1, gw2, gb2,
                 cvec):
    b = gathered.shape[0]
    blk = 2048
    grid = (b // blk,)

    def full(shape):
        return pl.BlockSpec(shape, lambda *_: (0,) * len(shape))

    return pl.pallas_call(
        _dense_kernel,
        grid=grid,
        in_specs=[
            pl.BlockSpec((blk, N_FEAT * EMB), lambda i: (i, 0)),
            full(w1.shape), full(b1.shape), full(w2.shape), full(b2.shape),
            full(wd.shape), full(bd.shape), full(gw1a.shape), full(gw1b.shape),
            full(gb1.shape), full(gw2.shape), full(gb2.shape), full(cvec.shape),
        ],
        out_specs=pl.BlockSpec((blk, 3), lambda i: (i, 0)),
        out_shape=jax.ShapeDtypeStruct((b, 3), jnp.float32),
    )(gathered, w1, b1, w2, b2, wd, bd, gw1a, gw1b, gb1, gw2, gb2, cvec)


def kernel(sdk_type, remote_host, device_type, dtu, click_goods_num,
           buy_click_num, goods_show_num, goods_click_num, brand_name,
           ppnet_cate1, ppnet_cate2, epnet_cate1, epnet_cate2,
           emb_table, gate_params, mlp_params):
    del epnet_cate1, epnet_cate2
    idx_t = jnp.concatenate(
        [sdk_type, remote_host, device_type, dtu, click_goods_num,
         buy_click_num, goods_show_num, goods_click_num, brand_name,
         ppnet_cate1, ppnet_cate2], axis=1).T
    b = sdk_type.shape[0]

    gathered = _sc_gather(emb_table, idx_t, batch=b, emb=EMB)

    w1 = jnp.concatenate([gp[0] for gp in gate_params], axis=1)
    b1 = jnp.concatenate([gp[1] for gp in gate_params]).reshape(1, -1)
    w2 = jax.scipy.linalg.block_diag(*[gp[2] for gp in gate_params])
    b2 = jnp.concatenate([gp[3] for gp in gate_params]).reshape(1, -1)

    wd = jnp.concatenate([mp['dense'][2][0] for mp in mlp_params], axis=1)
    bd = jnp.concatenate([mp['dense'][2][1] for mp in mlp_params]).reshape(1, -1)
    gw1 = jnp.concatenate([mp['gates'][2][0] for mp in mlp_params], axis=1)
    gw1a, gw1b = gw1[:MAIN_DIM], gw1[MAIN_DIM:]
    gb1 = jnp.concatenate([mp['gates'][2][1] for mp in mlp_params]).reshape(1, -1)
    gw2 = jax.scipy.linalg.block_diag(*[mp['gates'][2][2] for mp in mlp_params])
    gb2 = jnp.concatenate([mp['gates'][2][3] for mp in mlp_params]).reshape(1, -1)
    cvec = jnp.stack([mp['C'][2] for mp in mlp_params]).reshape(1, -1)

    out = _dense_stage(gathered, w1, b1, w2, b2, wd, bd, gw1a, gw1b, gb1,
                       gw2, gb2, cvec)
    return out.reshape(b, 3, 1)
